# baseline (speedup 1.0000x reference)
import functools

import jax
import jax.numpy as jnp
from jax import lax
from jax.experimental import pallas as pl
from jax.experimental.pallas import tpu as pltpu
from jax.experimental.pallas import tpu_sc as plsc

B = 128
L = 4096
D = 64
DP = 128
NC = 2
NS = 16
NW = NC * NS
C = L // NW
LG = C // 16
NBUF = 4
SBUF = 2
AHEAD = 2


def _body(x_hbm, pos_hbm, emb_hbm, pe_hbm, out_hbm,
          idx_all, pe_t, rows, srows, sem_g, sem_s):
    wid = lax.axis_index("s") * NC + lax.axis_index("c")
    l0 = wid * C

    pltpu.sync_copy(pos_hbm.at[pl.ds(l0, C)], idx_all.at[0])
    pltpu.async_copy(pe_hbm.at[idx_all.at[0]], rows.at[0], sem_g[0]).wait()

    lanes = lax.iota(jnp.int32, 16)

    def pe_tr(d, c2):
        dsplat = jnp.full((16,), 0, jnp.int32) + d
        for lg in range(LG):
            pe_t[d, pl.ds(lg * 16, 16)] = plsc.load_gather(
                rows.at[0], [lanes + lg * 16, dsplat])
        return c2

    lax.fori_loop(0, D, pe_tr, 0)

    pltpu.sync_copy(x_hbm.at[:, pl.ds(l0, C)], idx_all)

    def gather(b, q):
        pltpu.async_copy(emb_hbm.at[idx_all.at[b]], rows.at[q], sem_g[q])

    def wait_gather(b, q):
        pltpu.make_async_copy(emb_hbm.at[idx_all.at[b]], rows.at[q], sem_g[q]).wait()

    def store(b, s):
        pltpu.async_copy(srows.at[s], out_hbm.at[b, :, pl.ds(l0, C)], sem_s[s])

    def wait_store(b, s):
        pltpu.make_async_copy(srows.at[s], out_hbm.at[b, :, pl.ds(l0, C)],
                              sem_s[s]).wait()

    for b in range(AHEAD):
        gather(b, b % NBUF)

    def ring_body(t, carry):
        for p in range(NBUF):
            b = NBUF * t + p
            s = p % SBUF
            nb = b + AHEAD
            q2 = (p + AHEAD) % NBUF

            @pl.when(nb < B)
            def _issue():
                gather(nb, q2)

            wait_gather(b, p)

            @pl.when(b >= SBUF)
            def _drain():
                wait_store(b - SBUF, s)

            def d_body(d, c2):
                dsplat = jnp.full((16,), 0, jnp.int32) + d
                for lg in range(LG):
                    sl = pl.ds(lg * 16, 16)
                    srows[s, d, sl] = plsc.load_gather(
                        rows.at[p], [lanes + lg * 16, dsplat]) + pe_t[d, sl]
                return c2

            lax.fori_loop(0, D, d_body, 0)
            store(b, s)
        return carry

    lax.fori_loop(0, B // NBUF, ring_body, 0)

    for s in range(SBUF):
        wait_store(B - SBUF + s, s)


@jax.jit
def _run(x, emb_pad, pe_pad, pos_flat):
    mesh = plsc.VectorSubcoreMesh(core_axis_name="c", subcore_axis_name="s")
    f = functools.partial(
        pl.kernel,
        out_type=jax.ShapeDtypeStruct((B, D, L), jnp.float32),
        mesh=mesh,
        scratch_types=[
            pltpu.VMEM((B, C), jnp.int32),
            pltpu.VMEM((D, C), jnp.float32),
            pltpu.VMEM((NBUF, C, DP), jnp.float32),
            pltpu.VMEM((SBUF, D, C), jnp.float32),
            [pltpu.SemaphoreType.DMA] * NBUF,
            [pltpu.SemaphoreType.DMA] * SBUF,
        ],
        compiler_params=pltpu.CompilerParams(use_tc_tiling_on_sc=True,
                                             needs_layout_passes=False),
    )(_body)
    return f(x, pos_flat, emb_pad, pe_pad)


def kernel(x, emb_table, pe_table, positional_tokens):
    pad = ((0, 0), (0, DP - D))
    out_t = _run(x, jnp.pad(emb_table, pad), jnp.pad(pe_table, pad),
                 positional_tokens.reshape(L))
    return jnp.swapaxes(out_t, 1, 2)


# speedup vs baseline: 2.1456x; 2.1456x over previous
import functools

import jax
import jax.numpy as jnp
from jax import lax
from jax.experimental import pallas as pl
from jax.experimental.pallas import tpu as pltpu
from jax.experimental.pallas import tpu_sc as plsc

B = 128
L = 4096
D = 64
DP = 128
NC = 2
NS = 16
NW = NC * NS
C = L // NW
CB = C // 8
NBUF = 2
SBUF = 2
AHEAD = 1


def _body(x_hbm, pos_hbm, emb_hbm, pe_hbm, out_hbm,
          idx_all, pe_v, rows, srows, emb_sp, sem_g, sem_s):
    wid = lax.axis_index("s") * NC + lax.axis_index("c")
    l0 = wid * C
    blk0 = wid * CB

    @pl.when(lax.axis_index("s") == 0)
    def _stage_emb():
        pltpu.sync_copy(emb_hbm, emb_sp)

    plsc.subcore_barrier()

    pltpu.sync_copy(pos_hbm.at[pl.ds(l0, C)], idx_all.at[0])
    pltpu.async_copy(pe_hbm.at[idx_all.at[0]], rows.at[0], sem_g[0]).wait()

    def pe_pack(i, c2):
        for r in range(2):
            for j in range(D // 16):
                pe_v[i, pl.ds(r * D + j * 16, 16)] = rows[0, i * 2 + r, pl.ds(j * 16, 16)]
        return c2

    lax.fori_loop(0, C // 2, pe_pack, 0)

    pltpu.sync_copy(x_hbm.at[:, pl.ds(l0, C)], idx_all)

    def gather(b, q):
        pltpu.async_copy(emb_sp.at[idx_all.at[b]], rows.at[q], sem_g[q])

    def wait_gather(b, q):
        pltpu.make_async_copy(emb_sp.at[idx_all.at[b]], rows.at[q], sem_g[q]).wait()

    def store(b, s):
        pltpu.async_copy(srows.at[s], out_hbm.at[b, pl.ds(blk0, CB)], sem_s[s])

    def wait_store(b, s):
        pltpu.make_async_copy(srows.at[s], out_hbm.at[b, pl.ds(blk0, CB)],
                              sem_s[s]).wait()

    for b in range(AHEAD):
        gather(b, b % NBUF)

    def ring_body(t, carry):
        for p in range(NBUF):
            b = NBUF * t + p
            s = p % SBUF
            nb = b + AHEAD
            q2 = (p + AHEAD) % NBUF

            @pl.when(nb < B)
            def _issue():
                gather(nb, q2)

            wait_gather(b, p)

            @pl.when(b >= SBUF)
            def _drain():
                wait_store(b - SBUF, s)

            def blk_body(g, c2):
                for r in range(8):
                    for j in range(D // 16):
                        sl = pl.ds(j * 16, 16)
                        pesl = pl.ds((r % 2) * D + j * 16, 16)
                        srows[s, g, r, sl] = (rows[p, g * 8 + r, sl]
                                              + pe_v[g * 4 + r // 2, pesl])
                return c2

            lax.fori_loop(0, CB, blk_body, 0)
            store(b, s)
        return carry

    lax.fori_loop(0, B // NBUF, ring_body, 0)

    for s in range(SBUF):
        wait_store(B - SBUF + s, s)


@jax.jit
def _run(x, emb_pad, pe_pad, pos_flat):
    mesh = plsc.VectorSubcoreMesh(core_axis_name="c", subcore_axis_name="s")
    f = functools.partial(
        pl.kernel,
        out_type=jax.ShapeDtypeStruct((B, L // 8, 8, D), jnp.float32),
        mesh=mesh,
        scratch_types=[
            pltpu.VMEM((B, C), jnp.int32),
            pltpu.VMEM((C // 2, DP), jnp.float32),
            pltpu.VMEM((NBUF, C, DP), jnp.float32),
            pltpu.VMEM((SBUF, CB, 8, D), jnp.float32),
            pltpu.VMEM_SHARED((L, DP), jnp.float32),
            [pltpu.SemaphoreType.DMA] * NBUF,
            [pltpu.SemaphoreType.DMA] * SBUF,
        ],
        compiler_params=pltpu.CompilerParams(use_tc_tiling_on_sc=True),
    )(_body)
    return f(x, pos_flat, emb_pad, pe_pad)


def kernel(x, emb_table, pe_table, positional_tokens):
    pad = ((0, 0), (0, DP - D))
    out = _run(x, jnp.pad(emb_table, pad), jnp.pad(pe_table, pad),
               positional_tokens.reshape(L))
    return out.reshape(B, L, D)


# speedup vs baseline: 2.1557x; 1.0047x over previous
import functools

import jax
import jax.numpy as jnp
from jax import lax
from jax.experimental import pallas as pl
from jax.experimental.pallas import tpu as pltpu
from jax.experimental.pallas import tpu_sc as plsc

B = 128
L = 4096
D = 64
DP = 128
NC = 2
NS = 16
NW = NC * NS
C = L // NW
CB = C // 8
NBUF = 2
SBUF = 2
AHEAD = 1


def _body(x_hbm, pos_hbm, emb_hbm, pe_hbm, out_hbm,
          idx_all, pe_v, rows, srows, sem_g, sem_s):
    wid = lax.axis_index("s") * NC + lax.axis_index("c")
    l0 = wid * C
    blk0 = wid * CB


    pltpu.sync_copy(pos_hbm.at[pl.ds(l0, C)], idx_all.at[0])
    pltpu.async_copy(pe_hbm.at[idx_all.at[0]], rows.at[0], sem_g[0]).wait()

    def pe_pack(i, c2):
        for r in range(2):
            for j in range(D // 16):
                pe_v[i, pl.ds(r * D + j * 16, 16)] = rows[0, i * 2 + r, pl.ds(j * 16, 16)]
        return c2

    lax.fori_loop(0, C // 2, pe_pack, 0)

    pltpu.sync_copy(x_hbm.at[:, pl.ds(l0, C)], idx_all)

    def gather(b, q):
        pltpu.async_copy(emb_hbm.at[idx_all.at[b]], rows.at[q], sem_g[q])

    def wait_gather(b, q):
        pltpu.make_async_copy(emb_hbm.at[idx_all.at[b]], rows.at[q], sem_g[q]).wait()

    def store(b, s):
        pltpu.async_copy(srows.at[s], out_hbm.at[b, pl.ds(blk0, CB)], sem_s[s])

    def wait_store(b, s):
        pltpu.make_async_copy(srows.at[s], out_hbm.at[b, pl.ds(blk0, CB)],
                              sem_s[s]).wait()

    for b in range(AHEAD):
        gather(b, b % NBUF)

    def ring_body(t, carry):
        for p in range(NBUF):
            b = NBUF * t + p
            s = p % SBUF
            nb = b + AHEAD
            q2 = (p + AHEAD) % NBUF

            @pl.when(nb < B)
            def _issue():
                gather(nb, q2)

            wait_gather(b, p)

            @pl.when(b >= SBUF)
            def _drain():
                wait_store(b - SBUF, s)

            def blk_body(g, c2):
                for r in range(8):
                    for j in range(D // 16):
                        sl = pl.ds(j * 16, 16)
                        pesl = pl.ds((r % 2) * D + j * 16, 16)
                        srows[s, g, r, sl] = (rows[p, g * 8 + r, sl]
                                              + pe_v[g * 4 + r // 2, pesl])
                return c2

            lax.fori_loop(0, CB, blk_body, 0)
            store(b, s)
        return carry

    lax.fori_loop(0, B // NBUF, ring_body, 0)

    for s in range(SBUF):
        wait_store(B - SBUF + s, s)


@jax.jit
def _run(x, emb_pad, pe_pad, pos_flat):
    mesh = plsc.VectorSubcoreMesh(core_axis_name="c", subcore_axis_name="s")
    f = functools.partial(
        pl.kernel,
        out_type=jax.ShapeDtypeStruct((B, L // 8, 8, D), jnp.float32),
        mesh=mesh,
        scratch_types=[
            pltpu.VMEM((B, C), jnp.int32),
            pltpu.VMEM((C // 2, DP), jnp.float32),
            pltpu.VMEM((NBUF, C, DP), jnp.float32),
            pltpu.VMEM((SBUF, CB, 8, D), jnp.float32),
            [pltpu.SemaphoreType.DMA] * NBUF,
            [pltpu.SemaphoreType.DMA] * SBUF,
        ],
        compiler_params=pltpu.CompilerParams(use_tc_tiling_on_sc=True),
    )(_body)
    return f(x, pos_flat, emb_pad, pe_pad)


def kernel(x, emb_table, pe_table, positional_tokens):
    pad = ((0, 0), (0, DP - D))
    out = _run(x, jnp.pad(emb_table, pad), jnp.pad(pe_table, pad),
               positional_tokens.reshape(L))
    return out.reshape(B, L, D)


# speedup vs baseline: 2.7266x; 1.2648x over previous
import functools

import jax
import jax.numpy as jnp
from jax import lax
from jax.experimental import pallas as pl
from jax.experimental.pallas import tpu as pltpu
from jax.experimental.pallas import tpu_sc as plsc

B = 128
L = 4096
D = 64
DP = 128
NC = 2
NS = 16
NW = NC * NS
C = L // NW
CB = C // 8
NBUF = 4
SBUF = 2
AHEAD = 2


def _body(x_hbm, pos_hbm, emb_hbm, pe_hbm, out_hbm,
          idx_all, pe_v, rows, srows, sem_g, sem_s):
    wid = lax.axis_index("s") * NC + lax.axis_index("c")
    l0 = wid * C
    blk0 = wid * CB

    pltpu.sync_copy(pos_hbm.at[pl.ds(l0, C)], idx_all.at[0])
    pltpu.async_copy(pe_hbm.at[idx_all.at[0]], rows.at[0], sem_g[0]).wait()

    def pe_copy(i, c2):
        for j in range(D // 16):
            sl = pl.ds(j * 16, 16)
            pe_v[i, sl] = rows[0, i, sl]
        return c2

    lax.fori_loop(0, C, pe_copy, 0, unroll=2)

    pltpu.sync_copy(x_hbm.at[:, pl.ds(l0, C)], idx_all)

    def gather(b, q):
        pltpu.async_copy(emb_hbm.at[idx_all.at[b]], rows.at[q], sem_g[q])

    def wait_gather(b, q):
        pltpu.make_async_copy(emb_hbm.at[idx_all.at[b]], rows.at[q], sem_g[q]).wait()

    def store(b, s):
        pltpu.async_copy(srows.at[s], out_hbm.at[b, pl.ds(blk0, CB)], sem_s[s])

    def wait_store(b, s):
        pltpu.make_async_copy(srows.at[s], out_hbm.at[b, pl.ds(blk0, CB)],
                              sem_s[s]).wait()

    for b in range(AHEAD):
        gather(b, b % NBUF)

    def ring_body(t, carry):
        for p in range(NBUF):
            b = NBUF * t + p
            s = p % SBUF
            nb = b + AHEAD
            q2 = (p + AHEAD) % NBUF

            @pl.when(nb < B)
            def _issue():
                gather(nb, q2)

            wait_gather(b, p)

            @pl.when(b >= SBUF)
            def _drain():
                wait_store(b - SBUF, s)

            def blk_body(g, c2):
                for r in range(8):
                    for j in range(D // 16):
                        sl = pl.ds(j * 16, 16)
                        srows[s, g, r, sl] = rows[p, g * 8 + r, sl] + pe_v[g * 8 + r, sl]
                return c2

            lax.fori_loop(0, CB, blk_body, 0)
            store(b, s)
        return carry

    lax.fori_loop(0, B // NBUF, ring_body, 0)

    for s in range(SBUF):
        wait_store(B - SBUF + s, s)


@jax.jit
def _run(x, emb_pad, pe_pad, pos_flat):
    mesh = plsc.VectorSubcoreMesh(core_axis_name="c", subcore_axis_name="s")
    f = functools.partial(
        pl.kernel,
        out_type=jax.ShapeDtypeStruct((B, L // 8, 8, D), jnp.float32),
        mesh=mesh,
        scratch_types=[
            pltpu.VMEM((B, C), jnp.int32),
            pltpu.VMEM((C, D), jnp.float32),
            pltpu.VMEM((NBUF, C, DP), jnp.float32),
            pltpu.VMEM((SBUF, CB, 8, D), jnp.float32),
            [pltpu.SemaphoreType.DMA] * NBUF,
            [pltpu.SemaphoreType.DMA] * SBUF,
        ],
        compiler_params=pltpu.CompilerParams(use_tc_tiling_on_sc=True),
    )(_body)
    return f(x, pos_flat, emb_pad, pe_pad)


def kernel(x, emb_table, pe_table, positional_tokens):
    pad = ((0, 0), (0, DP - D))
    out = _run(x, jnp.pad(emb_table, pad), jnp.pad(pe_table, pad),
               positional_tokens.reshape(L))
    return out.reshape(B, L, D)


# speedup vs baseline: 2.7313x; 1.0017x over previous
import functools

import jax
import jax.numpy as jnp
from jax import lax
from jax.experimental import pallas as pl
from jax.experimental.pallas import tpu as pltpu
from jax.experimental.pallas import tpu_sc as plsc

B = 128
L = 4096
D = 64
DP = 128
NC = 2
NS = 16
NW = NC * NS
C = L // NW
CB = C // 8
NBUF = 4
SBUF = 2
AHEAD = 3


def _body(x_hbm, pos_hbm, emb_hbm, pe_hbm, out_hbm,
          idx_all, pe_v, rows, srows, sem_g, sem_s):
    wid = lax.axis_index("s") * NC + lax.axis_index("c")
    l0 = wid * C
    blk0 = wid * CB

    pltpu.sync_copy(pos_hbm.at[pl.ds(l0, C)], idx_all.at[0])
    pltpu.async_copy(pe_hbm.at[idx_all.at[0]], rows.at[0], sem_g[0]).wait()

    def pe_copy(i, c2):
        for j in range(D // 16):
            sl = pl.ds(j * 16, 16)
            pe_v[i, sl] = rows[0, i, sl]
        return c2

    lax.fori_loop(0, C, pe_copy, 0, unroll=2)

    pltpu.sync_copy(x_hbm.at[:, pl.ds(l0, C)], idx_all)

    def gather(b, q):
        pltpu.async_copy(emb_hbm.at[idx_all.at[b]], rows.at[q], sem_g[q])

    def wait_gather(b, q):
        pltpu.make_async_copy(emb_hbm.at[idx_all.at[b]], rows.at[q], sem_g[q]).wait()

    def store(b, s):
        pltpu.async_copy(srows.at[s], out_hbm.at[b, pl.ds(blk0, CB)], sem_s[s])

    def wait_store(b, s):
        pltpu.make_async_copy(srows.at[s], out_hbm.at[b, pl.ds(blk0, CB)],
                              sem_s[s]).wait()

    for b in range(AHEAD):
        gather(b, b % NBUF)

    def ring_body(t, carry):
        for p in range(NBUF):
            b = NBUF * t + p
            s = p % SBUF
            nb = b + AHEAD
            q2 = (p + AHEAD) % NBUF

            @pl.when(nb < B)
            def _issue():
                gather(nb, q2)

            wait_gather(b, p)

            @pl.when(b >= SBUF)
            def _drain():
                wait_store(b - SBUF, s)

            def blk_body(g, c2):
                for r in range(8):
                    for j in range(D // 16):
                        sl = pl.ds(j * 16, 16)
                        srows[s, g, r, sl] = rows[p, g * 8 + r, sl] + pe_v[g * 8 + r, sl]
                return c2

            lax.fori_loop(0, CB, blk_body, 0)
            store(b, s)
        return carry

    lax.fori_loop(0, B // NBUF, ring_body, 0)

    for s in range(SBUF):
        wait_store(B - SBUF + s, s)


@jax.jit
def _run(x, emb_pad, pe_pad, pos_flat):
    mesh = plsc.VectorSubcoreMesh(core_axis_name="c", subcore_axis_name="s")
    f = functools.partial(
        pl.kernel,
        out_type=jax.ShapeDtypeStruct((B, L // 8, 8, D), jnp.float32),
        mesh=mesh,
        scratch_types=[
            pltpu.VMEM((B, C), jnp.int32),
            pltpu.VMEM((C, D), jnp.float32),
            pltpu.VMEM((NBUF, C, DP), jnp.float32),
            pltpu.VMEM((SBUF, CB, 8, D), jnp.float32),
            [pltpu.SemaphoreType.DMA] * NBUF,
            [pltpu.SemaphoreType.DMA] * SBUF,
        ],
        compiler_params=pltpu.CompilerParams(use_tc_tiling_on_sc=True),
    )(_body)
    return f(x, pos_flat, emb_pad, pe_pad)


def kernel(x, emb_table, pe_table, positional_tokens):
    pad = ((0, 0), (0, DP - D))
    out = _run(x, jnp.pad(emb_table, pad), jnp.pad(pe_table, pad),
               positional_tokens.reshape(L))
    return out.reshape(B, L, D)
